# baseline (speedup 1.0000x reference)
import functools

import jax
import jax.numpy as jnp
from jax.experimental import pallas as pl
from jax.experimental.pallas import tpu as pltpu

_VMEM_LIMIT = 60 * 1024 * 1024


def _pick_tm(rows, limit):
    tm = min(rows, limit)
    while rows % tm:
        tm //= 2
    return tm


def _relu_stage_kernel(do_norm, m_ref, x_ref, o_ref):
    y = jnp.dot(m_ref[...], x_ref[...], preferred_element_type=jnp.float32)
    y = jnp.maximum(y, 0.0)
    if do_norm:
        sq = jnp.sum(y * y, axis=1, keepdims=True)
        y = y * jax.lax.rsqrt(jnp.maximum(sq, 1e-24))
    o_ref[...] = y.astype(o_ref.dtype)


def _relu_stage(m, x, *, do_norm, out_dtype, tm):
    r, k = m.shape
    d = x.shape[1]
    return pl.pallas_call(
        functools.partial(_relu_stage_kernel, do_norm),
        out_shape=jax.ShapeDtypeStruct((r, d), out_dtype),
        grid=(r // tm,),
        in_specs=[pl.BlockSpec((tm, k), lambda i: (i, 0)),
                  pl.BlockSpec((k, d), lambda i: (0, 0))],
        out_specs=pl.BlockSpec((tm, d), lambda i: (i, 0)),
        compiler_params=pltpu.CompilerParams(
            dimension_semantics=("parallel",),
            vmem_limit_bytes=_VMEM_LIMIT),
    )(m, x)


def _final_stage_kernel(m_ref, h1b_ref, x_ref, xv_ref, h1_ref, o_ref):
    y = jnp.dot(m_ref[...], h1b_ref[...], preferred_element_type=jnp.float32)
    y = jnp.maximum(y, 0.0)
    sq = jnp.sum(y * y, axis=1, keepdims=True)
    h2 = y * jax.lax.rsqrt(jnp.maximum(sq, 1e-24))

    x = x_ref[...]
    a = xv_ref[...]
    b = h1_ref[...]
    tot = (jnp.sum(x * x, axis=1, keepdims=True)
           + jnp.sum(a * a, axis=1, keepdims=True)
           + jnp.sum(b * b, axis=1, keepdims=True)
           + jnp.sum(h2 * h2, axis=1, keepdims=True))
    inv = jax.lax.rsqrt(jnp.maximum(tot, 1e-24))
    d = x.shape[1]
    o_ref[:, 0:d] = x * inv
    o_ref[:, d:2 * d] = a * inv
    o_ref[:, 2 * d:3 * d] = b * inv
    o_ref[:, 3 * d:4 * d] = h2 * inv


def _final_stage(m, h1b, x, xv, h1, *, tm):
    r, k = m.shape
    d = x.shape[1]
    row_spec = pl.BlockSpec((tm, d), lambda i: (i, 0))
    return pl.pallas_call(
        _final_stage_kernel,
        out_shape=jax.ShapeDtypeStruct((r, 4 * d), jnp.float32),
        grid=(r // tm,),
        in_specs=[pl.BlockSpec((tm, k), lambda i: (i, 0)),
                  pl.BlockSpec((k, d), lambda i: (0, 0)),
                  row_spec, row_spec, row_spec],
        out_specs=pl.BlockSpec((tm, 4 * d), lambda i: (i, 0)),
        compiler_params=pltpu.CompilerParams(
            dimension_semantics=("parallel",),
            vmem_limit_bytes=_VMEM_LIMIT),
    )(m, h1b, x, xv, h1)


def _build_operators(edge, size_v, size_e):
    row0, row1, row2 = edge[0], edge[1], edge[2]

    deg_e = jnp.zeros((size_e,), jnp.float32).at[row1].add(1.0)
    deg_v = jnp.zeros((size_v,), jnp.float32).at[row0].add(1.0)
    deg_g = jnp.zeros((size_v,), jnp.float32).at[row2].add(1.0)

    m_v2e = jnp.zeros((size_e, size_v), jnp.bfloat16).at[row1, row0].add(
        (1.0 / deg_e[row1]).astype(jnp.bfloat16))
    m_e2v = jnp.zeros((size_v, size_e), jnp.bfloat16).at[row0, row1].add(
        (1.0 / deg_v[row0]).astype(jnp.bfloat16))
    m_gcn = jnp.zeros((size_v, size_v), jnp.bfloat16).at[row2, row0].add(
        (1.0 / deg_g[row2]).astype(jnp.bfloat16))

    return m_v2e, m_e2v, m_gcn


def _graph_forward(x, edge, size_v, size_e):
    m_v2e, m_e2v, m_gcn = _build_operators(edge, size_v, size_e)
    x32 = x.astype(jnp.float32)
    xb = x32.astype(jnp.bfloat16)

    x_e = _relu_stage(m_v2e, xb, do_norm=False, out_dtype=jnp.bfloat16,
                      tm=_pick_tm(size_e, 512))
    x_v = _relu_stage(m_e2v, x_e, do_norm=True, out_dtype=jnp.float32,
                      tm=_pick_tm(size_v, 1024))
    h1 = _relu_stage(m_gcn, xb, do_norm=True, out_dtype=jnp.float32,
                     tm=_pick_tm(size_v, 1024))
    return _final_stage(m_gcn, h1.astype(jnp.bfloat16), x32, x_v, h1,
                        tm=_pick_tm(size_v, 512))


@jax.jit
def kernel(x1, x2, edge1, edge2):
    y1 = _graph_forward(x1, edge1, 8192, 2048)
    y2 = _graph_forward(x2, edge2, 6144, 1536)
    return y1, y2


# speedup vs baseline: 3.1616x; 3.1616x over previous
import functools

import jax
import jax.numpy as jnp
from jax.experimental import pallas as pl
from jax.experimental.pallas import tpu as pltpu

_VMEM_LIMIT = 60 * 1024 * 1024


def _pick_tm(rows, limit):
    tm = min(rows, limit)
    while rows % tm:
        tm //= 2
    return tm


def _relu_stage_kernel(do_norm, m_ref, x_ref, o_ref):
    y = jnp.dot(m_ref[...].astype(jnp.bfloat16), x_ref[...],
                preferred_element_type=jnp.float32)
    y = jnp.maximum(y, 0.0)
    if do_norm:
        sq = jnp.sum(y * y, axis=1, keepdims=True)
        y = y * jax.lax.rsqrt(jnp.maximum(sq, 1e-24))
    o_ref[...] = y.astype(o_ref.dtype)


def _relu_stage(m, x, *, do_norm, out_dtype, tm):
    r, k = m.shape
    d = x.shape[1]
    return pl.pallas_call(
        functools.partial(_relu_stage_kernel, do_norm),
        out_shape=jax.ShapeDtypeStruct((r, d), out_dtype),
        grid=(r // tm,),
        in_specs=[pl.BlockSpec((tm, k), lambda i: (i, 0)),
                  pl.BlockSpec((k, d), lambda i: (0, 0))],
        out_specs=pl.BlockSpec((tm, d), lambda i: (i, 0)),
        compiler_params=pltpu.CompilerParams(
            dimension_semantics=("parallel",),
            vmem_limit_bytes=_VMEM_LIMIT),
    )(m, x)


def _final_stage_kernel(m_ref, h1b_ref, x_ref, xv_ref, h1_ref, o_ref):
    y = jnp.dot(m_ref[...].astype(jnp.bfloat16), h1b_ref[...],
                preferred_element_type=jnp.float32)
    y = jnp.maximum(y, 0.0)
    sq = jnp.sum(y * y, axis=1, keepdims=True)
    h2 = y * jax.lax.rsqrt(jnp.maximum(sq, 1e-24))

    x = x_ref[...]
    a = xv_ref[...]
    b = h1_ref[...]
    tot = (jnp.sum(x * x, axis=1, keepdims=True)
           + jnp.sum(a * a, axis=1, keepdims=True)
           + jnp.sum(b * b, axis=1, keepdims=True)
           + jnp.sum(h2 * h2, axis=1, keepdims=True))
    inv = jax.lax.rsqrt(jnp.maximum(tot, 1e-24))
    d = x.shape[1]
    o_ref[:, 0:d] = x * inv
    o_ref[:, d:2 * d] = a * inv
    o_ref[:, 2 * d:3 * d] = b * inv
    o_ref[:, 3 * d:4 * d] = h2 * inv


def _final_stage(m, h1b, x, xv, h1, *, tm):
    r, k = m.shape
    d = x.shape[1]
    row_spec = pl.BlockSpec((tm, d), lambda i: (i, 0))
    return pl.pallas_call(
        _final_stage_kernel,
        out_shape=jax.ShapeDtypeStruct((r, 4 * d), jnp.float32),
        grid=(r // tm,),
        in_specs=[pl.BlockSpec((tm, k), lambda i: (i, 0)),
                  pl.BlockSpec((k, d), lambda i: (0, 0)),
                  row_spec, row_spec, row_spec],
        out_specs=pl.BlockSpec((tm, 4 * d), lambda i: (i, 0)),
        compiler_params=pltpu.CompilerParams(
            dimension_semantics=("parallel",),
            vmem_limit_bytes=_VMEM_LIMIT),
    )(m, h1b, x, xv, h1)


def _build_operators(edge, size_v, size_e):
    row0, row1, row2 = edge[0], edge[1], edge[2]

    deg_e = jnp.zeros((size_e,), jnp.float32).at[row1].add(1.0)
    deg_v = jnp.zeros((size_v,), jnp.float32).at[row0].add(1.0)
    deg_g = jnp.zeros((size_v,), jnp.float32).at[row2].add(1.0)

    m_v2e = jnp.zeros((size_e, size_v), jnp.float32).at[row1, row0].add(
        1.0 / deg_e[row1])
    m_e2v = jnp.zeros((size_v, size_e), jnp.float32).at[row0, row1].add(
        1.0 / deg_v[row0])
    m_gcn = jnp.zeros((size_v, size_v), jnp.float32).at[row2, row0].add(
        1.0 / deg_g[row2])

    return m_v2e, m_e2v, m_gcn


def _graph_forward(x, edge, size_v, size_e):
    m_v2e, m_e2v, m_gcn = _build_operators(edge, size_v, size_e)
    x32 = x.astype(jnp.float32)
    xb = x32.astype(jnp.bfloat16)

    x_e = _relu_stage(m_v2e, xb, do_norm=False, out_dtype=jnp.bfloat16,
                      tm=_pick_tm(size_e, 512))
    x_v = _relu_stage(m_e2v, x_e, do_norm=True, out_dtype=jnp.float32,
                      tm=_pick_tm(size_v, 1024))
    h1 = _relu_stage(m_gcn, xb, do_norm=True, out_dtype=jnp.float32,
                     tm=_pick_tm(size_v, 512))
    return _final_stage(m_gcn, h1.astype(jnp.bfloat16), x32, x_v, h1,
                        tm=_pick_tm(size_v, 512))


@jax.jit
def kernel(x1, x2, edge1, edge2):
    y1 = _graph_forward(x1, edge1, 8192, 2048)
    y2 = _graph_forward(x2, edge2, 6144, 1536)
    return y1, y2


# speedup vs baseline: 5.7348x; 1.8139x over previous
import functools

import jax
import jax.numpy as jnp
from jax.experimental import pallas as pl
from jax.experimental.pallas import tpu as pltpu

_VMEM_LIMIT = 60 * 1024 * 1024


def _pick_tm(rows, limit):
    tm = min(rows, limit)
    while rows % tm:
        tm //= 2
    return tm


def _relu_stage_kernel(do_norm, m_ref, x_ref, o_ref):
    y = jnp.dot(m_ref[...].astype(jnp.bfloat16), x_ref[...],
                preferred_element_type=jnp.float32)
    y = jnp.maximum(y, 0.0)
    if do_norm:
        sq = jnp.sum(y * y, axis=1, keepdims=True)
        y = y * jax.lax.rsqrt(jnp.maximum(sq, 1e-24))
    o_ref[...] = y.astype(o_ref.dtype)


def _relu_stage(m, x, *, do_norm, out_dtype, tm):
    r, k = m.shape
    d = x.shape[1]
    return pl.pallas_call(
        functools.partial(_relu_stage_kernel, do_norm),
        out_shape=jax.ShapeDtypeStruct((r, d), out_dtype),
        grid=(r // tm,),
        in_specs=[pl.BlockSpec((tm, k), lambda i: (i, 0)),
                  pl.BlockSpec((k, d), lambda i: (0, 0))],
        out_specs=pl.BlockSpec((tm, d), lambda i: (i, 0)),
        compiler_params=pltpu.CompilerParams(
            dimension_semantics=("parallel",),
            vmem_limit_bytes=_VMEM_LIMIT),
    )(m, x)


def _relu_stage_t_kernel(m_ref, x_ref, o_ref):
    y = jax.lax.dot_general(m_ref[...].astype(jnp.bfloat16), x_ref[...],
                            (((0,), (0,)), ((), ())),
                            preferred_element_type=jnp.float32)
    y = jnp.maximum(y, 0.0)
    sq = jnp.sum(y * y, axis=1, keepdims=True)
    o_ref[...] = y * jax.lax.rsqrt(jnp.maximum(sq, 1e-24))


def _relu_stage_t(m, x, *, tm):
    k, r = m.shape
    d = x.shape[1]
    return pl.pallas_call(
        _relu_stage_t_kernel,
        out_shape=jax.ShapeDtypeStruct((r, d), jnp.float32),
        grid=(r // tm,),
        in_specs=[pl.BlockSpec((k, tm), lambda i: (0, i)),
                  pl.BlockSpec((k, d), lambda i: (0, 0))],
        out_specs=pl.BlockSpec((tm, d), lambda i: (i, 0)),
        compiler_params=pltpu.CompilerParams(
            dimension_semantics=("parallel",),
            vmem_limit_bytes=_VMEM_LIMIT),
    )(m, x)


def _final_stage_kernel(m_ref, h1b_ref, x_ref, xv_ref, h1_ref, o_ref):
    y = jnp.dot(m_ref[...].astype(jnp.bfloat16), h1b_ref[...],
                preferred_element_type=jnp.float32)
    y = jnp.maximum(y, 0.0)
    sq = jnp.sum(y * y, axis=1, keepdims=True)
    h2 = y * jax.lax.rsqrt(jnp.maximum(sq, 1e-24))

    x = x_ref[...]
    a = xv_ref[...]
    b = h1_ref[...]
    tot = (jnp.sum(x * x, axis=1, keepdims=True)
           + jnp.sum(a * a, axis=1, keepdims=True)
           + jnp.sum(b * b, axis=1, keepdims=True)
           + jnp.sum(h2 * h2, axis=1, keepdims=True))
    inv = jax.lax.rsqrt(jnp.maximum(tot, 1e-24))
    d = x.shape[1]
    o_ref[:, 0:d] = x * inv
    o_ref[:, d:2 * d] = a * inv
    o_ref[:, 2 * d:3 * d] = b * inv
    o_ref[:, 3 * d:4 * d] = h2 * inv


def _final_stage(m, h1b, x, xv, h1, *, tm):
    r, k = m.shape
    d = x.shape[1]
    row_spec = pl.BlockSpec((tm, d), lambda i: (i, 0))
    return pl.pallas_call(
        _final_stage_kernel,
        out_shape=jax.ShapeDtypeStruct((r, 4 * d), jnp.float32),
        grid=(r // tm,),
        in_specs=[pl.BlockSpec((tm, k), lambda i: (i, 0)),
                  pl.BlockSpec((k, d), lambda i: (0, 0)),
                  row_spec, row_spec, row_spec],
        out_specs=pl.BlockSpec((tm, 4 * d), lambda i: (i, 0)),
        compiler_params=pltpu.CompilerParams(
            dimension_semantics=("parallel",),
            vmem_limit_bytes=_VMEM_LIMIT),
    )(m, h1b, x, xv, h1)


def _graph_forward(x, edge, size_v, size_e):
    row0, row1, row2 = edge[0], edge[1], edge[2]

    a_t = jnp.zeros((size_e, size_v), jnp.float32).at[row1, row0].add(1.0)
    b = jnp.zeros((size_v, size_v), jnp.float32).at[row2, row0].add(1.0)

    x32 = x.astype(jnp.float32)
    xb = x32.astype(jnp.bfloat16)

    z = _relu_stage(a_t, xb, do_norm=False, out_dtype=jnp.float32,
                    tm=_pick_tm(size_e, 512))
    deg_e = jnp.sum(a_t, axis=1)
    zb = (z * (1.0 / jnp.maximum(deg_e, 1.0))[:, None]).astype(jnp.bfloat16)

    x_v = _relu_stage_t(a_t, zb, tm=_pick_tm(size_v, 1024))
    h1 = _relu_stage(b, xb, do_norm=True, out_dtype=jnp.float32,
                     tm=_pick_tm(size_v, 512))
    return _final_stage(b, h1.astype(jnp.bfloat16), x32, x_v, h1,
                        tm=_pick_tm(size_v, 512))


@jax.jit
def kernel(x1, x2, edge1, edge2):
    y1 = _graph_forward(x1, edge1, 8192, 2048)
    y2 = _graph_forward(x2, edge2, 6144, 1536)
    return y1, y2


# speedup vs baseline: 6.0600x; 1.0567x over previous
import functools

import jax
import jax.numpy as jnp
from jax.experimental import pallas as pl
from jax.experimental.pallas import tpu as pltpu

_VMEM_LIMIT = 60 * 1024 * 1024


def _pick_tm(rows, limit):
    tm = min(rows, limit)
    while rows % tm:
        tm //= 2
    return tm


def _relu_stage_kernel(do_norm, m_ref, x_ref, o_ref):
    y = jnp.dot(m_ref[...], x_ref[...], preferred_element_type=jnp.float32)
    y = jnp.maximum(y, 0.0)
    if do_norm:
        sq = jnp.sum(y * y, axis=1, keepdims=True)
        y = y * jax.lax.rsqrt(jnp.maximum(sq, 1e-24))
    o_ref[...] = y.astype(o_ref.dtype)


def _relu_stage(m, x, *, do_norm, out_dtype, tm):
    r, k = m.shape
    d = x.shape[1]
    return pl.pallas_call(
        functools.partial(_relu_stage_kernel, do_norm),
        out_shape=jax.ShapeDtypeStruct((r, d), out_dtype),
        grid=(r // tm,),
        in_specs=[pl.BlockSpec((tm, k), lambda i: (i, 0)),
                  pl.BlockSpec((k, d), lambda i: (0, 0))],
        out_specs=pl.BlockSpec((tm, d), lambda i: (i, 0)),
        compiler_params=pltpu.CompilerParams(
            dimension_semantics=("parallel",),
            vmem_limit_bytes=_VMEM_LIMIT),
    )(m, x)


def _relu_stage_t_kernel(m_ref, x_ref, o_ref):
    y = jax.lax.dot_general(m_ref[...], x_ref[...],
                            (((0,), (0,)), ((), ())),
                            preferred_element_type=jnp.float32)
    y = jnp.maximum(y, 0.0)
    sq = jnp.sum(y * y, axis=1, keepdims=True)
    o_ref[...] = y * jax.lax.rsqrt(jnp.maximum(sq, 1e-24))


def _relu_stage_t(m, x, *, tm):
    k, r = m.shape
    d = x.shape[1]
    return pl.pallas_call(
        _relu_stage_t_kernel,
        out_shape=jax.ShapeDtypeStruct((r, d), jnp.float32),
        grid=(r // tm,),
        in_specs=[pl.BlockSpec((k, tm), lambda i: (0, i)),
                  pl.BlockSpec((k, d), lambda i: (0, 0))],
        out_specs=pl.BlockSpec((tm, d), lambda i: (i, 0)),
        compiler_params=pltpu.CompilerParams(
            dimension_semantics=("parallel",),
            vmem_limit_bytes=_VMEM_LIMIT),
    )(m, x)


def _final_stage_kernel(m_ref, h1b_ref, x_ref, xv_ref, h1_ref, o_ref):
    y = jnp.dot(m_ref[...], h1b_ref[...], preferred_element_type=jnp.float32)
    y = jnp.maximum(y, 0.0)
    sq = jnp.sum(y * y, axis=1, keepdims=True)
    h2 = y * jax.lax.rsqrt(jnp.maximum(sq, 1e-24))

    x = x_ref[...]
    a = xv_ref[...]
    b = h1_ref[...]
    tot = (jnp.sum(x * x, axis=1, keepdims=True)
           + jnp.sum(a * a, axis=1, keepdims=True)
           + jnp.sum(b * b, axis=1, keepdims=True)
           + jnp.sum(h2 * h2, axis=1, keepdims=True))
    inv = jax.lax.rsqrt(jnp.maximum(tot, 1e-24))
    d = x.shape[1]
    o_ref[:, 0:d] = x * inv
    o_ref[:, d:2 * d] = a * inv
    o_ref[:, 2 * d:3 * d] = b * inv
    o_ref[:, 3 * d:4 * d] = h2 * inv


def _final_stage(m, h1b, x, xv, h1, *, tm):
    r, k = m.shape
    d = x.shape[1]
    row_spec = pl.BlockSpec((tm, d), lambda i: (i, 0))
    return pl.pallas_call(
        _final_stage_kernel,
        out_shape=jax.ShapeDtypeStruct((r, 4 * d), jnp.float32),
        grid=(r // tm,),
        in_specs=[pl.BlockSpec((tm, k), lambda i: (i, 0)),
                  pl.BlockSpec((k, d), lambda i: (0, 0)),
                  row_spec, row_spec, row_spec],
        out_specs=pl.BlockSpec((tm, 4 * d), lambda i: (i, 0)),
        compiler_params=pltpu.CompilerParams(
            dimension_semantics=("parallel",),
            vmem_limit_bytes=_VMEM_LIMIT),
    )(m, h1b, x, xv, h1)


def _graph_forward(x, edge, size_v, size_e):
    row0, row1, row2 = edge[0], edge[1], edge[2]

    a_t32 = jnp.zeros((size_e, size_v), jnp.float32).at[row1, row0].add(1.0)
    b32 = jnp.zeros((size_v, size_v), jnp.float32).at[row2, row0].add(1.0)
    a_t = a_t32.astype(jnp.bfloat16)
    b = b32.astype(jnp.bfloat16)

    x32 = x.astype(jnp.float32)
    xb = x32.astype(jnp.bfloat16)

    z = _relu_stage(a_t, xb, do_norm=False, out_dtype=jnp.float32,
                    tm=_pick_tm(size_e, 512))
    deg_e = jnp.sum(a_t, axis=1, dtype=jnp.float32)
    zb = (z * (1.0 / jnp.maximum(deg_e, 1.0))[:, None]).astype(jnp.bfloat16)

    x_v = _relu_stage_t(a_t, zb, tm=_pick_tm(size_v, 1024))
    h1 = _relu_stage(b, xb, do_norm=True, out_dtype=jnp.float32,
                     tm=_pick_tm(size_v, 512))
    return _final_stage(b, h1.astype(jnp.bfloat16), x32, x_v, h1,
                        tm=_pick_tm(size_v, 512))


@jax.jit
def kernel(x1, x2, edge1, edge2):
    y1 = _graph_forward(x1, edge1, 8192, 2048)
    y2 = _graph_forward(x2, edge2, 6144, 1536)
    return y1, y2
